# baseline (speedup 1.0000x reference)
import jax
import jax.numpy as jnp
from jax import lax
from jax.experimental import pallas as pl
from jax.experimental.pallas import tpu as pltpu
from jax.experimental.pallas import tpu_sc as plsc

H = 512
N = H * H
NC, NS, L = 2, 16, 16
NW = NC * NS
CHUNK = N // NW
VECS = CHUNK // L
NHIST = 9
HB = 256
HALLOC = HB * L
OUTW = NHIST * HB
PX_UNROLL = 4


def _hist_body(inp_hbm, tar_hbm, ma_hbm, mb_hbm, out_hbm,
               inp_v, tar_v, ma_v, mb_v,
               h0, h1, h2, h3, h4, h5, h6, h7, h8, out_v, sem):
    hs = [h0, h1, h2, h3, h4, h5, h6, h7, h8]
    wid = lax.axis_index("s") * NC + lax.axis_index("c")
    base = wid * CHUNK

    copies = []
    for c in range(3):
        copies.append(pltpu.async_copy(
            inp_hbm.at[pl.ds(c * N + base, CHUNK)],
            inp_v.at[pl.ds(c * CHUNK, CHUNK)], sem))
        copies.append(pltpu.async_copy(
            tar_hbm.at[pl.ds(c * N + base, CHUNK)],
            tar_v.at[pl.ds(c * CHUNK, CHUNK)], sem))
    copies.append(pltpu.async_copy(ma_hbm.at[pl.ds(base, CHUNK)], ma_v, sem))
    copies.append(pltpu.async_copy(mb_hbm.at[pl.ds(base, CHUNK)], mb_v, sem))

    zeros = jnp.zeros((L,), jnp.float32)

    def zero_body(j, carry):
        for h in hs:
            h[pl.ds(j * L, L)] = zeros
        return carry

    lax.fori_loop(0, HALLOC // L, zero_body, 0)

    for cp in copies:
        cp.wait()

    lane = lax.iota(jnp.int32, L)

    def px_body(i, carry):
        for u in range(PX_UNROLL):
            off = (i * PX_UNROLL + u) * L
            m = ma_v[pl.ds(off, L)]
            mb = mb_v[pl.ds(off, L)]
            for c in range(3):
                x = inp_v[pl.ds(c * CHUNK + off, L)]
                v = jnp.minimum(jnp.maximum((x + 1.0) * 0.5, 0.0), 1.0) * 255.0
                idx = v.astype(jnp.int32) * L + lane
                plsc.addupdate_scatter(hs[c], [idx], m)
                plsc.addupdate_scatter(hs[3 + c], [idx], v * m)
                y = tar_v[pl.ds(c * CHUNK + off, L)]
                w = jnp.minimum(jnp.maximum((y + 1.0) * 0.5, 0.0), 1.0) * 255.0
                idx2 = w.astype(jnp.int32) * L + lane
                plsc.addupdate_scatter(hs[6 + c], [idx2], mb)
        return carry

    lax.fori_loop(0, VECS // PX_UNROLL, px_body, 0)

    rotbases = [lane * L + ((lane + l) & (L - 1)) for l in range(L)]

    def red_body(j, carry):
        for k in range(NHIST):
            acc = plsc.load_gather(hs[k], [rotbases[0] + j * (L * L)])
            for l in range(1, L):
                acc = acc + plsc.load_gather(hs[k], [rotbases[l] + j * (L * L)])
            out_v[pl.ds(k * HB + j * L, L)] = acc
        return carry

    lax.fori_loop(0, HB // L, red_body, 0)

    pltpu.sync_copy(out_v, out_hbm.at[pl.ds(wid * OUTW, OUTW)])


def _make_hist_call(interpret=False):
    mesh = plsc.VectorSubcoreMesh(core_axis_name="c", subcore_axis_name="s",
                                  num_cores=NC, num_subcores=NS)
    return pl.kernel(
        _hist_body,
        out_type=jax.ShapeDtypeStruct((NW * OUTW,), jnp.float32),
        mesh=mesh,
        scratch_types=[
            pltpu.VMEM((3 * CHUNK,), jnp.float32),
            pltpu.VMEM((3 * CHUNK,), jnp.float32),
            pltpu.VMEM((CHUNK,), jnp.float32),
            pltpu.VMEM((CHUNK,), jnp.float32),
        ] + [pltpu.VMEM((HALLOC,), jnp.float32) for _ in range(NHIST)] + [
            pltpu.VMEM((OUTW,), jnp.float32),
            pltpu.SemaphoreType.DMA,
        ],
        compiler_params=pltpu.CompilerParams(needs_layout_passes=False),
        interpret=interpret,
    )


def kernel(input_data, target_data, mask_src, mask_tar):
    inp = input_data.reshape(3 * N)
    tar = target_data.reshape(3 * N)
    ma = mask_src.reshape(N)
    mb = mask_tar.reshape(N)

    parts = _make_hist_call()(inp, tar, ma, mb)
    hists = parts.reshape(NW, NHIST, HB).sum(axis=0)

    dst_cnt = hists[0:3]
    dst_sum = hists[3:6]
    ref_cnt = hists[6:9]

    cdf_d = jnp.stack([jnp.cumsum(dst_cnt[c] / jnp.sum(dst_cnt[c]))
                       for c in range(3)])
    cdf_r = jnp.stack([jnp.cumsum(ref_cnt[c] / jnp.sum(ref_cnt[c]))
                       for c in range(3)])

    cond = ((cdf_d[:, 1:, None] >= cdf_r[:, None, 0:255])
            & (cdf_d[:, 1:, None] <= cdf_r[:, None, 1:256]))
    any_c = jnp.any(cond, axis=2)
    first_j = jnp.argmax(cond, axis=2) + 1
    vals = jnp.where(any_c, first_j, jnp.arange(1, 256)[None, :])
    table = jnp.concatenate(
        [jnp.zeros((3, 1), vals.dtype), vals], axis=1).at[:, 255].set(255)
    t = table.astype(jnp.float32)

    return jnp.sum(jnp.abs(dst_sum - dst_cnt * t)) / jnp.float32(3 * N)


# speedup vs baseline: 1.0484x; 1.0484x over previous
import jax
import jax.numpy as jnp
from jax import lax
from jax.experimental import pallas as pl
from jax.experimental.pallas import tpu as pltpu
from jax.experimental.pallas import tpu_sc as plsc

H = 512
N = H * H
NC, NS, L = 2, 16, 16
NW = NC * NS
CHUNK = N // NW
VECS = CHUNK // L
NHIST = 9
HB = 256
HALLOC = HB * L
OUTW = NHIST * HB
PX_UNROLL = 4


def _hist_body(inp_hbm, tar_hbm, ma_hbm, mb_hbm, out_hbm,
               inp_v, tar_v, ma_v, mb_v,
               h0, h1, h2, h3, h4, h5, h6, h7, h8, out_v, sem):
    hs = [h0, h1, h2, h3, h4, h5, h6, h7, h8]
    wid = lax.axis_index("s") * NC + lax.axis_index("c")
    base = wid * CHUNK

    copies = []
    for c in range(3):
        copies.append(pltpu.async_copy(
            inp_hbm.at[pl.ds(c * N + base, CHUNK)],
            inp_v.at[pl.ds(c * CHUNK, CHUNK)], sem))
        copies.append(pltpu.async_copy(
            tar_hbm.at[pl.ds(c * N + base, CHUNK)],
            tar_v.at[pl.ds(c * CHUNK, CHUNK)], sem))
    copies.append(pltpu.async_copy(ma_hbm.at[pl.ds(base, CHUNK)], ma_v, sem))
    copies.append(pltpu.async_copy(mb_hbm.at[pl.ds(base, CHUNK)], mb_v, sem))

    zeros = jnp.zeros((L,), jnp.float32)

    def zero_body(j, carry):
        for h in hs:
            h[pl.ds(j * L, L)] = zeros
        return carry

    lax.fori_loop(0, HALLOC // L, zero_body, 0)

    for cp in copies:
        cp.wait()

    lane = lax.iota(jnp.int32, L)

    ones = jnp.ones((L,), jnp.float32)

    def px_body(i, carry):
        for u in range(PX_UNROLL):
            off = (i * PX_UNROLL + u) * L
            m = ma_v[pl.ds(off, L)] > 0.0
            mb = mb_v[pl.ds(off, L)] > 0.0
            for c in range(3):
                x = inp_v[pl.ds(c * CHUNK + off, L)]
                v = jnp.minimum(jnp.maximum((x + 1.0) * 127.5, 0.0), 255.0)
                idx = v.astype(jnp.int32) * L + lane
                plsc.addupdate_scatter(hs[c], [idx], ones, mask=m)
                plsc.addupdate_scatter(hs[3 + c], [idx], v, mask=m)
                y = tar_v[pl.ds(c * CHUNK + off, L)]
                w = jnp.minimum(jnp.maximum((y + 1.0) * 127.5, 0.0), 255.0)
                idx2 = w.astype(jnp.int32) * L + lane
                plsc.addupdate_scatter(hs[6 + c], [idx2], ones, mask=mb)
        return carry

    lax.fori_loop(0, VECS // PX_UNROLL, px_body, 0)

    rotbases = [lane * L + ((lane + l) & (L - 1)) for l in range(L)]

    def red_body(j, carry):
        for k in range(NHIST):
            acc = plsc.load_gather(hs[k], [rotbases[0] + j * (L * L)])
            for l in range(1, L):
                acc = acc + plsc.load_gather(hs[k], [rotbases[l] + j * (L * L)])
            out_v[pl.ds(k * HB + j * L, L)] = acc
        return carry

    lax.fori_loop(0, HB // L, red_body, 0)

    pltpu.sync_copy(out_v, out_hbm.at[pl.ds(wid * OUTW, OUTW)])


def _make_hist_call(interpret=False):
    mesh = plsc.VectorSubcoreMesh(core_axis_name="c", subcore_axis_name="s",
                                  num_cores=NC, num_subcores=NS)
    return pl.kernel(
        _hist_body,
        out_type=jax.ShapeDtypeStruct((NW * OUTW,), jnp.float32),
        mesh=mesh,
        scratch_types=[
            pltpu.VMEM((3 * CHUNK,), jnp.float32),
            pltpu.VMEM((3 * CHUNK,), jnp.float32),
            pltpu.VMEM((CHUNK,), jnp.float32),
            pltpu.VMEM((CHUNK,), jnp.float32),
        ] + [pltpu.VMEM((HALLOC,), jnp.float32) for _ in range(NHIST)] + [
            pltpu.VMEM((OUTW,), jnp.float32),
            pltpu.SemaphoreType.DMA,
        ],
        compiler_params=pltpu.CompilerParams(needs_layout_passes=False),
        interpret=interpret,
    )


def kernel(input_data, target_data, mask_src, mask_tar):
    inp = input_data.reshape(3 * N)
    tar = target_data.reshape(3 * N)
    ma = mask_src.reshape(N)
    mb = mask_tar.reshape(N)

    parts = _make_hist_call()(inp, tar, ma, mb)
    hists = parts.reshape(NW, NHIST, HB).sum(axis=0)

    dst_cnt = hists[0:3]
    dst_sum = hists[3:6]
    ref_cnt = hists[6:9]

    cdf_d = jnp.stack([jnp.cumsum(dst_cnt[c] / jnp.sum(dst_cnt[c]))
                       for c in range(3)])
    cdf_r = jnp.stack([jnp.cumsum(ref_cnt[c] / jnp.sum(ref_cnt[c]))
                       for c in range(3)])

    cond = ((cdf_d[:, 1:, None] >= cdf_r[:, None, 0:255])
            & (cdf_d[:, 1:, None] <= cdf_r[:, None, 1:256]))
    any_c = jnp.any(cond, axis=2)
    first_j = jnp.argmax(cond, axis=2) + 1
    vals = jnp.where(any_c, first_j, jnp.arange(1, 256)[None, :])
    table = jnp.concatenate(
        [jnp.zeros((3, 1), vals.dtype), vals], axis=1).at[:, 255].set(255)
    t = table.astype(jnp.float32)

    return jnp.sum(jnp.abs(dst_sum - dst_cnt * t)) / jnp.float32(3 * N)


# speedup vs baseline: 1.1177x; 1.0661x over previous
import jax
import jax.numpy as jnp
from jax import lax
from jax.experimental import pallas as pl
from jax.experimental.pallas import tpu as pltpu
from jax.experimental.pallas import tpu_sc as plsc

H = 512
N = H * H
NC, NS, L = 2, 16, 16
NW = NC * NS
CHUNK = N // NW
ROWS = H // NW
VECS = CHUNK // L
NHIST = 9
HB = 256
HALLOC = HB * L
OUTW = NHIST * HB
PX_UNROLL = 4


def _hist_body(inp_hbm, tar_hbm, ma_hbm, mb_hbm, out_hbm,
               inp_v, tar_v, ma_v, mb_v,
               h0, h1, h2, h3, h4, h5, h6, h7, h8, out_v, sem):
    hs = [h0, h1, h2, h3, h4, h5, h6, h7, h8]
    wid = lax.axis_index("s") * NC + lax.axis_index("c")
    r0 = wid * ROWS

    copies = []
    for c in range(3):
        copies.append(pltpu.async_copy(
            inp_hbm.at[0, c, pl.ds(r0, ROWS), :], inp_v.at[c], sem))
        copies.append(pltpu.async_copy(
            tar_hbm.at[0, c, pl.ds(r0, ROWS), :], tar_v.at[c], sem))
    copies.append(pltpu.async_copy(ma_hbm.at[0, 0, pl.ds(r0, ROWS), :],
                                   ma_v, sem))
    copies.append(pltpu.async_copy(mb_hbm.at[0, 0, pl.ds(r0, ROWS), :],
                                   mb_v, sem))

    zeros = jnp.zeros((L,), jnp.float32)

    def zero_body(j, carry):
        for h in hs:
            h[pl.ds(j * L, L)] = zeros
        return carry

    lax.fori_loop(0, HALLOC // L, zero_body, 0)

    for cp in copies:
        cp.wait()

    lane = lax.iota(jnp.int32, L)

    ones = jnp.ones((L,), jnp.float32)

    def px_body(i, carry):
        for u in range(PX_UNROLL):
            g = i * PX_UNROLL + u
            r = g >> 5
            col = (g & 31) * L
            m = ma_v[r, pl.ds(col, L)] > 0.0
            mb = mb_v[r, pl.ds(col, L)] > 0.0
            for c in range(3):
                x = inp_v[c, r, pl.ds(col, L)]
                v = jnp.minimum(jnp.maximum((x + 1.0) * 127.5, 0.0), 255.0)
                idx = v.astype(jnp.int32) * L + lane
                plsc.addupdate_scatter(hs[c], [idx], ones, mask=m)
                plsc.addupdate_scatter(hs[3 + c], [idx], v, mask=m)
                y = tar_v[c, r, pl.ds(col, L)]
                w = jnp.minimum(jnp.maximum((y + 1.0) * 127.5, 0.0), 255.0)
                idx2 = w.astype(jnp.int32) * L + lane
                plsc.addupdate_scatter(hs[6 + c], [idx2], ones, mask=mb)
        return carry

    lax.fori_loop(0, VECS // PX_UNROLL, px_body, 0)

    rotbases = [lane * L + ((lane + l) & (L - 1)) for l in range(L)]

    def red_body(j, carry):
        for k in range(NHIST):
            acc = plsc.load_gather(hs[k], [rotbases[0] + j * (L * L)])
            for l in range(1, L):
                acc = acc + plsc.load_gather(hs[k], [rotbases[l] + j * (L * L)])
            out_v[pl.ds(k * HB + j * L, L)] = acc
        return carry

    lax.fori_loop(0, HB // L, red_body, 0)

    pltpu.sync_copy(out_v, out_hbm.at[pl.ds(wid * OUTW, OUTW)])


def _make_hist_call(interpret=False):
    mesh = plsc.VectorSubcoreMesh(core_axis_name="c", subcore_axis_name="s",
                                  num_cores=NC, num_subcores=NS)
    return pl.kernel(
        _hist_body,
        out_type=jax.ShapeDtypeStruct((NW * OUTW,), jnp.float32),
        mesh=mesh,
        scratch_types=[
            pltpu.VMEM((3, ROWS, H), jnp.float32),
            pltpu.VMEM((3, ROWS, H), jnp.float32),
            pltpu.VMEM((ROWS, H), jnp.float32),
            pltpu.VMEM((ROWS, H), jnp.float32),
        ] + [pltpu.VMEM((HALLOC,), jnp.float32) for _ in range(NHIST)] + [
            pltpu.VMEM((OUTW,), jnp.float32),
            pltpu.SemaphoreType.DMA,
        ],
        compiler_params=pltpu.CompilerParams(needs_layout_passes=False),
        interpret=interpret,
    )


def kernel(input_data, target_data, mask_src, mask_tar):
    parts = _make_hist_call()(input_data, target_data, mask_src, mask_tar)
    hists = parts.reshape(NW, NHIST, HB).sum(axis=0)

    dst_cnt = hists[0:3]
    dst_sum = hists[3:6]
    ref_cnt = hists[6:9]

    cdf_d = jnp.stack([jnp.cumsum(dst_cnt[c] / jnp.sum(dst_cnt[c]))
                       for c in range(3)])
    cdf_r = jnp.stack([jnp.cumsum(ref_cnt[c] / jnp.sum(ref_cnt[c]))
                       for c in range(3)])

    cond = ((cdf_d[:, 1:, None] >= cdf_r[:, None, 0:255])
            & (cdf_d[:, 1:, None] <= cdf_r[:, None, 1:256]))
    any_c = jnp.any(cond, axis=2)
    first_j = jnp.argmax(cond, axis=2) + 1
    vals = jnp.where(any_c, first_j, jnp.arange(1, 256)[None, :])
    table = jnp.concatenate(
        [jnp.zeros((3, 1), vals.dtype), vals], axis=1).at[:, 255].set(255)
    t = table.astype(jnp.float32)

    return jnp.sum(jnp.abs(dst_sum - dst_cnt * t)) / jnp.float32(3 * N)
